# baseline (speedup 1.0000x reference)
import jax
import jax.numpy as jnp
from jax import lax
from jax.experimental import pallas as pl
from jax.experimental.pallas import tpu as pltpu
from jax.experimental.pallas import tpu_sc as plsc

D = 64
V = 1000000
SCALE = 8.0
NC = 2
NS = 16
B = 4096
L = 200
D_PER_C = D // NC
B_PER_T = B // NS
HALF = 128
NIDX = L * HALF
CW = 62500
CLN = 62504
LANES = 16
RB = 8
LCH = 40
NLC = L // LCH
GN = LCH * HALF


def _emb_body(table_t, x_t, out_hbm,
              itmp, idx_a, idx_b, dbuf, sbuf, sprow, lsem, gsem, ssem):
    cid = lax.axis_index("c")
    sid = lax.axis_index("s")
    b0 = sid * B_PER_T
    idx_1d = (idx_a, idx_b)

    for h in range(2):
        @pl.loop(0, L // RB)
        def _(blk):
            pltpu.sync_copy(
                x_t.at[pl.ds(blk * RB, RB), pl.ds(b0 + h * HALF, HALF)],
                itmp)

            @pl.loop(0, RB)
            def _(i):
                base = pl.multiple_of((blk * RB + i) * HALF, 8)
                for c in range(HALF // LANES):
                    idx_1d[h][pl.ds(base + c * LANES, LANES)] = (
                        itmp[i, pl.ds(c * LANES, LANES)])

    def row_load_desc(d):
        off = pl.multiple_of(sid * CW - 4 * lax.rem(sid, 2), 8)
        return pltpu.make_async_copy(
            table_t.at[d, pl.ds(off, CLN)],
            sprow.at[pl.ds(off, CLN)],
            lsem,
        )

    def gather_desc(lc, h):
        return pltpu.make_async_copy(
            sprow.at[idx_1d[h].at[pl.ds(lc * GN, GN)]], dbuf, gsem)

    def store_desc(d, lc):
        ti = lax.div(d, 8)
        r = lax.rem(d, 8)
        return pltpu.make_async_copy(
            sbuf,
            out_hbm.at[pl.ds(lc * LCH, LCH), ti, pl.ds(2 * sid, 2), r, :],
            ssem)

    def scale_pack(h):
        @pl.loop(0, LCH, unroll=4)
        def _(i):
            base = pl.multiple_of(i * HALF, 8)
            for c in range(HALF // LANES):
                sbuf[i, h, pl.ds(c * LANES, LANES)] = (
                    dbuf[pl.ds(base + c * LANES, LANES)] * SCALE)

    d_base = cid * D_PER_C

    row_load_desc(d_base).start()
    row_load_desc(d_base).wait()
    plsc.subcore_barrier()

    @pl.loop(0, D_PER_C)
    def _(dl):
        d = d_base + dl

        for lc in range(NLC):
            if lc > 0:
                store_desc(d, lc - 1).wait()
            for h in range(2):
                gather_desc(lc, h).start()
                gather_desc(lc, h).wait()
                scale_pack(h)
            store_desc(d, lc).start()
        plsc.subcore_barrier()

        @pl.when(dl < D_PER_C - 1)
        def _():
            row_load_desc(d + 1).start()

        @pl.when(dl < D_PER_C - 1)
        def _():
            row_load_desc(d + 1).wait()
        store_desc(d, NLC - 1).wait()
        plsc.subcore_barrier()


@jax.jit
def _emb_lookup(table_tr, x_tr):
    mesh = plsc.VectorSubcoreMesh(core_axis_name="c", subcore_axis_name="s")
    f = pl.kernel(
        _emb_body,
        out_type=jax.ShapeDtypeStruct((L, 8, B // HALF, 8, HALF),
                                      jnp.float32),
        mesh=mesh,
        scratch_types=[
            pltpu.VMEM((RB, HALF), jnp.int32),
            pltpu.VMEM((NIDX,), jnp.int32),
            pltpu.VMEM((NIDX,), jnp.int32),
            pltpu.VMEM((GN,), jnp.float32),
            pltpu.VMEM((LCH, 2, HALF), jnp.float32),
            pltpu.VMEM_SHARED((V,), jnp.float32),
            pltpu.SemaphoreType.DMA,
            pltpu.SemaphoreType.DMA,
            pltpu.SemaphoreType.DMA,
        ],
        compiler_params=pltpu.CompilerParams(use_tc_tiling_on_sc=False),
    )
    return f(table_tr, x_tr)


def kernel(x, table):
    table_tr = jnp.transpose(table)
    x_tr = jnp.transpose(x)
    out5 = _emb_lookup(table_tr, x_tr)
    out = jnp.transpose(out5, (2, 4, 0, 1, 3)).reshape(B, L, D)
    return out
